# baseline (speedup 1.0000x reference)
import functools

import jax
import jax.numpy as jnp
from jax import lax
from jax.experimental import pallas as pl
from jax.experimental.pallas import tpu as pltpu
from jax.experimental.pallas import tpu_sc as plsc

EMB_DIM = 32
LANES = 16


def _mf_kernel_body(bpw, nc,
                    user_hbm, item_hbm, ut_hbm, it_hbm, ub_hbm, ib_hbm,
                    out_hbm,
                    uidx_v, iidx_v, urows_v, irows_v, ub_v, ib_v, out_v, sem):
    wid = lax.axis_index("s") * nc + lax.axis_index("c")
    base = wid * bpw

    pltpu.sync_copy(user_hbm.at[pl.ds(base, bpw)], uidx_v)
    pltpu.sync_copy(item_hbm.at[pl.ds(base, bpw)], iidx_v)

    cu = pltpu.async_copy(ut_hbm.at[uidx_v], urows_v, sem)
    ci = pltpu.async_copy(it_hbm.at[iidx_v], irows_v, sem)
    cub = pltpu.async_copy(ub_hbm.at[uidx_v], ub_v, sem)
    cib = pltpu.async_copy(ib_hbm.at[iidx_v], ib_v, sem)
    cu.wait()
    ci.wait()
    cub.wait()
    cib.wait()

    lane_ids = lax.iota(jnp.int32, LANES)

    def group(g, carry):
        sl = pl.ds(g * LANES, LANES)
        row_idx = lane_ids + g * LANES
        acc = ub_v[sl] + ib_v[sl]
        for d in range(EMB_DIM):
            dvec = jnp.full((LANES,), d, dtype=jnp.int32)
            u = plsc.load_gather(urows_v, [row_idx, dvec])
            iv = plsc.load_gather(irows_v, [row_idx, dvec])
            acc = acc + u * iv
        out_v[sl] = acc
        return carry

    lax.fori_loop(0, bpw // LANES, group, 0)

    pltpu.sync_copy(out_v, out_hbm.at[pl.ds(base, bpw)])


def kernel(user, item, user_table, item_table, user_bias_table, item_bias_table):
    batch = user.shape[0]
    info = plsc.get_sparse_core_info()
    nc, ns = info.num_cores, info.num_subcores
    nw = nc * ns
    bpw = batch // nw

    mesh = plsc.VectorSubcoreMesh(core_axis_name="c", subcore_axis_name="s")
    k = pl.kernel(
        functools.partial(_mf_kernel_body, bpw, nc),
        out_type=jax.ShapeDtypeStruct((batch,), jnp.float32),
        mesh=mesh,
        compiler_params=pltpu.CompilerParams(
            use_tc_tiling_on_sc=False, needs_layout_passes=False),
        scratch_types=[
            pltpu.VMEM((bpw,), jnp.int32),
            pltpu.VMEM((bpw,), jnp.int32),
            pltpu.VMEM((bpw, EMB_DIM), jnp.float32),
            pltpu.VMEM((bpw, EMB_DIM), jnp.float32),
            pltpu.VMEM((bpw,), jnp.float32),
            pltpu.VMEM((bpw,), jnp.float32),
            pltpu.VMEM((bpw,), jnp.float32),
            pltpu.SemaphoreType.DMA,
        ],
    )
    return k(user.astype(jnp.int32), item.astype(jnp.int32),
             user_table, item_table,
             user_bias_table.reshape(-1), item_bias_table.reshape(-1))
